# baseline (speedup 1.0000x reference)
import jax
import jax.numpy as jnp
from jax import lax
from jax.experimental import pallas as pl
from jax.experimental.pallas import tpu as pltpu
from jax.experimental.pallas import tpu_sc as plsc

N, D, G, NSEG = 320000, 128, 128, 10000
H1, H2 = 64, 256
H3, H4 = 32, 16

ROWS_BLK = 2560
NP = 327680
NB = NP // ROWS_BLK
NB_REAL = N // ROWS_BLK

NC, NS = 2, 16
NW = NC * NS
NGRP = NP // 128
GPW = NGRP // NW
KBUF = 2
T_OUT = GPW // KBUF
ACC_ROWS = 10112
ZROWS = ACC_ROWS // NS
W_TILES = 10
WROWS = NSEG // W_TILES


def _mlp1_body(x_ref, w1_ref, b1_ref, w2_ref, b2_ref, o_ref):
    x = x_ref[...].astype(jnp.bfloat16)
    h1 = jnp.maximum(
        jnp.dot(x, w1_ref[...].astype(jnp.bfloat16),
                preferred_element_type=jnp.float32) + b1_ref[...],
        0.0)
    h = jnp.dot(h1.astype(jnp.bfloat16), w2_ref[...].astype(jnp.bfloat16),
                preferred_element_type=jnp.float32) + b2_ref[...]
    gates = jax.nn.sigmoid(h[:, :G])
    o_ref[...] = h[:, G:] * gates


def _mlp1(node_states, W1, b1, W2, b2, interpret=False):
    return pl.pallas_call(
        _mlp1_body,
        grid=(NB,),
        in_specs=[
            pl.BlockSpec((ROWS_BLK, D), lambda i: (jnp.minimum(i, NB_REAL - 1), 0)),
            pl.BlockSpec((D, H1), lambda i: (0, 0)),
            pl.BlockSpec((1, H1), lambda i: (0, 0)),
            pl.BlockSpec((H1, H2), lambda i: (0, 0)),
            pl.BlockSpec((1, H2), lambda i: (0, 0)),
        ],
        out_specs=pl.BlockSpec((ROWS_BLK, G), lambda i: (i, 0)),
        out_shape=jax.ShapeDtypeStruct((NP, G), jnp.float32),
        interpret=interpret,
    )(node_states, W1, b1.reshape(1, H1), W2, b2.reshape(1, H2))


def _segsum_body(vals_hbm, idx_hbm, zeros_hbm, out_hbm, acc, chunk, idxb):
    c = lax.axis_index("c")
    s = lax.axis_index("s")
    pltpu.sync_copy(zeros_hbm, acc.at[pl.ds(s * ZROWS, ZROWS)])
    w = c * NS + s
    pltpu.sync_copy(idx_hbm.at[pl.ds(w * GPW, GPW)], idxb)
    plsc.subcore_barrier()

    def outer(t, carry):
        r0 = (w * GPW + t * KBUF) * 128
        pltpu.sync_copy(vals_hbm.at[pl.ds(r0, KBUF * 128)], chunk)
        for j in range(KBUF):
            pltpu.sync_copy(chunk.at[pl.ds(j * 128, 128)],
                            acc.at[idxb.at[t * KBUF + j]], add=True)
        return carry

    lax.fori_loop(0, T_OUT, outer, 0)
    plsc.subcore_barrier()

    @pl.when(s < W_TILES)
    def _():
        pltpu.sync_copy(acc.at[pl.ds(s * WROWS, WROWS)],
                        out_hbm.at[pl.ds(c * NSEG + s * WROWS, WROWS)])


def _segsum(vals, idx2d, zeros):
    mesh = plsc.VectorSubcoreMesh(
        core_axis_name="c", subcore_axis_name="s",
        num_cores=NC, num_subcores=NS)
    return pl.kernel(
        _segsum_body,
        out_type=jax.ShapeDtypeStruct((NC * NSEG, G), jnp.float32),
        mesh=mesh,
        scratch_types=[
            pltpu.VMEM_SHARED((ACC_ROWS, G), jnp.float32),
            pltpu.VMEM((KBUF * 128, G), jnp.float32),
            pltpu.VMEM((GPW, 128), jnp.int32),
        ],
    )(vals, idx2d, zeros)


def _mlp2_body(p_ref, w3_ref, b3_ref, w4_ref, b4_ref, o_ref):
    g = p_ref[:NSEG, :] + p_ref[NSEG:, :]
    h = jnp.maximum(
        jnp.dot(g, w3_ref[...], preferred_element_type=jnp.float32) + b3_ref[...],
        0.0)
    o_ref[...] = (
        jnp.dot(h, w4_ref[...], preferred_element_type=jnp.float32) + b4_ref[...])


def _mlp2(partials, W3, b3, W4, b4, interpret=False):
    return pl.pallas_call(
        _mlp2_body,
        out_shape=jax.ShapeDtypeStruct((NSEG, H4), jnp.float32),
        interpret=interpret,
    )(partials, W3, b3.reshape(1, H3), W4, b4.reshape(1, H4))


@jax.jit
def kernel(node_states, graph_idx, W1, b1, W2, b2, W3, b3, W4, b4):
    vals = _mlp1(node_states, W1, b1, W2, b2)
    idx2d = jnp.pad(graph_idx.astype(jnp.int32), (0, NP - N),
                    constant_values=NSEG).reshape(NGRP, 128)
    zeros = jnp.zeros((ZROWS, G), jnp.float32)
    partials = _segsum(vals, idx2d, zeros)
    return _mlp2(partials, W3, b3, W4, b4)


# speedup vs baseline: 1.1262x; 1.1262x over previous
import jax
import jax.numpy as jnp
from jax import lax
from jax.experimental import pallas as pl
from jax.experimental.pallas import tpu as pltpu
from jax.experimental.pallas import tpu_sc as plsc

N, D, G, NSEG = 320000, 128, 128, 10000
H1, H2 = 64, 256
H3, H4 = 32, 16

ROWS_BLK = 2560
NP = 327680
NB = NP // ROWS_BLK
NB_REAL = N // ROWS_BLK

NC, NS = 2, 16
NW = NC * NS
NGRP = NP // 128
GPW = NGRP // NW
KBUF = 2
T_OUT = GPW // KBUF
ACC_ROWS = 10112
ZROWS = ACC_ROWS // NS
W_TILES = 10
WROWS = NSEG // W_TILES


def _mlp1_body(x_ref, w1_ref, b1_ref, w2_ref, b2_ref, o_ref):
    x = x_ref[...].astype(jnp.bfloat16)
    h1 = jnp.maximum(
        jnp.dot(x, w1_ref[...].astype(jnp.bfloat16),
                preferred_element_type=jnp.float32) + b1_ref[...],
        0.0)
    h = jnp.dot(h1.astype(jnp.bfloat16), w2_ref[...].astype(jnp.bfloat16),
                preferred_element_type=jnp.float32) + b2_ref[...]
    gates = jax.nn.sigmoid(h[:, :G])
    o_ref[...] = h[:, G:] * gates


def _mlp1(node_states, W1, b1, W2, b2, interpret=False):
    return pl.pallas_call(
        _mlp1_body,
        grid=(NB,),
        in_specs=[
            pl.BlockSpec((ROWS_BLK, D), lambda i: (jnp.minimum(i, NB_REAL - 1), 0)),
            pl.BlockSpec((D, H1), lambda i: (0, 0)),
            pl.BlockSpec((1, H1), lambda i: (0, 0)),
            pl.BlockSpec((H1, H2), lambda i: (0, 0)),
            pl.BlockSpec((1, H2), lambda i: (0, 0)),
        ],
        out_specs=pl.BlockSpec((ROWS_BLK, G), lambda i: (i, 0)),
        out_shape=jax.ShapeDtypeStruct((NP, G), jnp.float32),
        interpret=interpret,
    )(node_states, W1, b1.reshape(1, H1), W2, b2.reshape(1, H2))


def _segsum_body(vals_hbm, idx_hbm, zeros_hbm, out_hbm, acc,
                 buf0, buf1, idxb, sem0, sem1):
    c = lax.axis_index("c")
    s = lax.axis_index("s")
    pltpu.sync_copy(zeros_hbm, acc.at[pl.ds(s * ZROWS, ZROWS)])
    w = c * NS + s
    pltpu.sync_copy(idx_hbm.at[pl.ds(w * GPW, GPW)], idxb)
    plsc.subcore_barrier()
    base = w * GPW

    def start_load(g, buf, sem):
        r = jnp.minimum(g, NGRP - 1) * 128
        pltpu.async_copy(vals_hbm.at[pl.ds(r, 128)], buf, sem)

    def wait_load(buf, sem):
        pltpu.make_async_copy(vals_hbm.at[pl.ds(0, 128)], buf, sem).wait()

    start_load(base, buf0, sem0)
    start_load(base + 1, buf1, sem1)

    def outer(t2, carry):
        g = base + 2 * t2
        wait_load(buf0, sem0)
        pltpu.sync_copy(buf0, acc.at[idxb.at[2 * t2]], add=True)
        start_load(g + 2, buf0, sem0)
        wait_load(buf1, sem1)
        pltpu.sync_copy(buf1, acc.at[idxb.at[2 * t2 + 1]], add=True)
        start_load(g + 3, buf1, sem1)
        return carry

    lax.fori_loop(0, GPW // 2, outer, 0)
    wait_load(buf0, sem0)
    wait_load(buf1, sem1)
    plsc.subcore_barrier()

    @pl.when(s < W_TILES)
    def _():
        pltpu.sync_copy(acc.at[pl.ds(s * WROWS, WROWS)],
                        out_hbm.at[pl.ds(c * NSEG + s * WROWS, WROWS)])


def _segsum(vals, idx2d, zeros):
    mesh = plsc.VectorSubcoreMesh(
        core_axis_name="c", subcore_axis_name="s",
        num_cores=NC, num_subcores=NS)
    return pl.kernel(
        _segsum_body,
        out_type=jax.ShapeDtypeStruct((NC * NSEG, G), jnp.float32),
        mesh=mesh,
        scratch_types=[
            pltpu.VMEM_SHARED((ACC_ROWS, G), jnp.float32),
            pltpu.VMEM((128, G), jnp.float32),
            pltpu.VMEM((128, G), jnp.float32),
            pltpu.VMEM((GPW, 128), jnp.int32),
            pltpu.SemaphoreType.DMA,
            pltpu.SemaphoreType.DMA,
        ],
    )(vals, idx2d, zeros)


def _mlp2_body(p_ref, w3_ref, b3_ref, w4_ref, b4_ref, o_ref):
    g = p_ref[:NSEG, :] + p_ref[NSEG:, :]
    h = jnp.maximum(
        jnp.dot(g, w3_ref[...], preferred_element_type=jnp.float32) + b3_ref[...],
        0.0)
    o_ref[...] = (
        jnp.dot(h, w4_ref[...], preferred_element_type=jnp.float32) + b4_ref[...])


def _mlp2(partials, W3, b3, W4, b4, interpret=False):
    return pl.pallas_call(
        _mlp2_body,
        out_shape=jax.ShapeDtypeStruct((NSEG, H4), jnp.float32),
        interpret=interpret,
    )(partials, W3, b3.reshape(1, H3), W4, b4.reshape(1, H4))


@jax.jit
def kernel(node_states, graph_idx, W1, b1, W2, b2, W3, b3, W4, b4):
    vals = _mlp1(node_states, W1, b1, W2, b2)
    idx2d = jnp.pad(graph_idx.astype(jnp.int32), (0, NP - N),
                    constant_values=NSEG).reshape(NGRP, 128)
    zeros = jnp.zeros((ZROWS, G), jnp.float32)
    partials = _segsum(vals, idx2d, zeros)
    return _mlp2(partials, W3, b3, W4, b4)


# speedup vs baseline: 1.2337x; 1.0954x over previous
import jax
import jax.numpy as jnp
from jax import lax
from jax.experimental import pallas as pl
from jax.experimental.pallas import tpu as pltpu
from jax.experimental.pallas import tpu_sc as plsc

N, D, G, NSEG = 320000, 128, 128, 10000
H1, H2 = 64, 256
H3, H4 = 32, 16

ROWS_BLK = 2560
NP = 327680
NB = NP // ROWS_BLK
NB_REAL = N // ROWS_BLK

NCHUNK = 2
NB_C = NB // NCHUNK
NGRP_C = NP // 128 // NCHUNK

NC, NS = 2, 16
NW = NC * NS
ACC_ROWS = 10112
ZROWS = ACC_ROWS // NS
W_TILES = 10
WROWS = NSEG // W_TILES


def _mlp1_body(x_ref, w1_ref, b1_ref, w2_ref, b2_ref, o_ref):
    x = x_ref[...].astype(jnp.bfloat16)
    h1 = jnp.maximum(
        jnp.dot(x, w1_ref[...].astype(jnp.bfloat16),
                preferred_element_type=jnp.float32) + b1_ref[...],
        0.0)
    h = jnp.dot(h1.astype(jnp.bfloat16), w2_ref[...].astype(jnp.bfloat16),
                preferred_element_type=jnp.float32) + b2_ref[...]
    gates = jax.nn.sigmoid(h[:, :G])
    o_ref[...] = h[:, G:] * gates


def _mlp1_chunk(k, node_states, W1, b1, W2, b2):
    return pl.pallas_call(
        _mlp1_body,
        grid=(NB_C,),
        in_specs=[
            pl.BlockSpec(
                (ROWS_BLK, D),
                lambda i: (jnp.minimum(k * NB_C + i, NB_REAL - 1), 0)),
            pl.BlockSpec((D, H1), lambda i: (0, 0)),
            pl.BlockSpec((1, H1), lambda i: (0, 0)),
            pl.BlockSpec((H1, H2), lambda i: (0, 0)),
            pl.BlockSpec((1, H2), lambda i: (0, 0)),
        ],
        out_specs=pl.BlockSpec((ROWS_BLK, G), lambda i: (i, 0)),
        out_shape=jax.ShapeDtypeStruct((NB_C * ROWS_BLK, G), jnp.float32),
        name=f"mlp1_chunk{k}",
    )(node_states, W1, b1.reshape(1, H1), W2, b2.reshape(1, H2))


def _segsum_body(vals_hbm, idx_hbm, zeros_hbm, out_hbm, acc,
                 buf0, buf1, idxb, sem0, sem1):
    c = lax.axis_index("c")
    s = lax.axis_index("s")
    gpw = NGRP_C // NW
    pltpu.sync_copy(zeros_hbm, acc.at[pl.ds(s * ZROWS, ZROWS)])
    w = c * NS + s
    pltpu.sync_copy(idx_hbm.at[pl.ds(w * gpw, gpw)], idxb)
    plsc.subcore_barrier()
    base = w * gpw

    def start_load(g, buf, sem):
        r = jnp.minimum(g, NGRP_C - 1) * 128
        pltpu.async_copy(vals_hbm.at[pl.ds(r, 128)], buf, sem)

    def wait_load(buf, sem):
        pltpu.make_async_copy(vals_hbm.at[pl.ds(0, 128)], buf, sem).wait()

    start_load(base, buf0, sem0)
    start_load(base + 1, buf1, sem1)

    def outer(t2, carry):
        g = base + 2 * t2
        wait_load(buf0, sem0)
        pltpu.sync_copy(buf0, acc.at[idxb.at[2 * t2]], add=True)
        start_load(g + 2, buf0, sem0)
        wait_load(buf1, sem1)
        pltpu.sync_copy(buf1, acc.at[idxb.at[2 * t2 + 1]], add=True)
        start_load(g + 3, buf1, sem1)
        return carry

    lax.fori_loop(0, gpw // 2, outer, 0)
    wait_load(buf0, sem0)
    wait_load(buf1, sem1)
    plsc.subcore_barrier()

    @pl.when(s < W_TILES)
    def _():
        pltpu.sync_copy(acc.at[pl.ds(s * WROWS, WROWS)],
                        out_hbm.at[pl.ds(c * NSEG + s * WROWS, WROWS)])


def _segsum_chunk(vals, idx2d, zeros):
    mesh = plsc.VectorSubcoreMesh(
        core_axis_name="c", subcore_axis_name="s",
        num_cores=NC, num_subcores=NS)
    return pl.kernel(
        _segsum_body,
        out_type=jax.ShapeDtypeStruct((NC * NSEG, G), jnp.float32),
        mesh=mesh,
        scratch_types=[
            pltpu.VMEM_SHARED((ACC_ROWS, G), jnp.float32),
            pltpu.VMEM((128, G), jnp.float32),
            pltpu.VMEM((128, G), jnp.float32),
            pltpu.VMEM((NGRP_C // NW, 128), jnp.int32),
            pltpu.SemaphoreType.DMA,
            pltpu.SemaphoreType.DMA,
        ],
    )(vals, idx2d, zeros)


def _mlp2_body(*refs):
    p_refs = refs[:NCHUNK]
    w3_ref, b3_ref, w4_ref, b4_ref, o_ref = refs[NCHUNK:]
    g = p_refs[0][:NSEG, :] + p_refs[0][NSEG:, :]
    for k in range(1, NCHUNK):
        g = g + p_refs[k][:NSEG, :] + p_refs[k][NSEG:, :]
    h = jnp.maximum(
        jnp.dot(g, w3_ref[...], preferred_element_type=jnp.float32) + b3_ref[...],
        0.0)
    o_ref[...] = (
        jnp.dot(h, w4_ref[...], preferred_element_type=jnp.float32) + b4_ref[...])


def _mlp2(partials, W3, b3, W4, b4):
    return pl.pallas_call(
        _mlp2_body,
        out_shape=jax.ShapeDtypeStruct((NSEG, H4), jnp.float32),
    )(*partials, W3, b3.reshape(1, H3), W4, b4.reshape(1, H4))


@jax.jit
def kernel(node_states, graph_idx, W1, b1, W2, b2, W3, b3, W4, b4):
    idx2d = jnp.pad(graph_idx.astype(jnp.int32), (0, NP - N),
                    constant_values=NSEG).reshape(NCHUNK, NGRP_C, 128)
    zeros = jnp.zeros((ZROWS, G), jnp.float32)
    partials = []
    for k in range(NCHUNK):
        vals_k = _mlp1_chunk(k, node_states, W1, b1, W2, b2)
        partials.append(_segsum_chunk(vals_k, idx2d[k], zeros))
    return _mlp2(partials, W3, b3, W4, b4)
